# Initial kernel scaffold; baseline (speedup 1.0000x reference)
import jax
import jax.numpy as jnp
from jax.experimental import pallas as pl


def kernel(observation, dones, executing_option, W_meta, b_meta, W_mv, b_mv, W_term, b_term, W1, b1, W2, b2, Wv, bv):
    raise NotImplementedError("write your pallas kernel here")


# speedup vs baseline: 1.8340x; 1.8340x over previous
import jax
import jax.numpy as jnp
from jax import lax
from jax.experimental import pallas as pl
from jax.experimental.pallas import tpu as pltpu

B = 1024
OBS = 1024
HID = 1024
E = 8
ACT = 16


def _fused_body(obs_ref, dones_ref, eo_ref, Wm_ref, bm_ref, Wmv_ref, bmv_ref,
                Wt_ref, bt_ref, W1_ref, b1_ref, W2_ref, b2_ref, Wv_ref, bv_ref,
                act_ref, val_ref, lp_ref, ma_ref, mv_ref, mlp_ref, tp_ref,
                newopt_ref):
    e = pl.program_id(0)

    @pl.when(e == 0)
    def _meta():
        obs = obs_ref[...]
        meta_logits = jnp.dot(obs, Wm_ref[...],
                              preferred_element_type=jnp.float32) + bm_ref[...]
        m = jnp.max(meta_logits, axis=-1, keepdims=True)
        idx = lax.broadcasted_iota(jnp.int32, (B, E), 1)
        meta_actions = jnp.min(
            jnp.where(meta_logits == m, idx, E), axis=-1, keepdims=True)
        s = jnp.sum(jnp.exp(meta_logits - m), axis=-1, keepdims=True)
        meta_log_probs = -jnp.log(s)
        meta_values = jnp.dot(obs, Wmv_ref[...],
                              preferred_element_type=jnp.float32) + bmv_ref[...]
        term_logits = jnp.dot(obs, Wt_ref[...],
                              preferred_element_type=jnp.float32) + bt_ref[...]
        tp_all = jax.nn.sigmoid(term_logits)
        eo = eo_ref[...]
        onehot_eo = (idx == eo)
        termination_probs = jnp.sum(
            jnp.where(onehot_eo, tp_all, 0.0), axis=-1, keepdims=True)
        terminates = (dones_ref[...] != 0) | (termination_probs > 0.5)
        new_option = jnp.where(terminates, meta_actions, eo)
        ma_ref[...] = meta_actions
        mv_ref[...] = meta_values
        mlp_ref[...] = meta_log_probs
        tp_ref[...] = termination_probs
        newopt_ref[...] = new_option

    obs = obs_ref[...]
    h = jnp.maximum(
        jnp.dot(obs, W1_ref[0], preferred_element_type=jnp.float32)
        + b1_ref[0], 0.0)
    logits = jnp.dot(h, W2_ref[0], preferred_element_type=jnp.float32) + b2_ref[0]
    vals = jnp.dot(h, Wv_ref[0], preferred_element_type=jnp.float32) + bv_ref[0]
    m = jnp.max(logits, axis=-1, keepdims=True)
    idx = lax.broadcasted_iota(jnp.int32, (B, ACT), 1)
    acts = jnp.min(jnp.where(logits == m, idx, ACT), axis=-1, keepdims=True)
    lps = -jnp.log(jnp.sum(jnp.exp(logits - m), axis=-1, keepdims=True))

    mask = newopt_ref[...] == e
    act_ref[...] = jnp.where(mask, acts, act_ref[...])
    val_ref[...] = jnp.where(mask, vals, val_ref[...])
    lp_ref[...] = jnp.where(mask, lps, lp_ref[...])


def kernel(observation, dones, executing_option, W_meta, b_meta, W_mv, b_mv,
           W_term, b_term, W1, b1, W2, b2, Wv, bv):
    dones_i = dones.astype(jnp.int32).reshape(B, 1)
    eo = executing_option.astype(jnp.int32).reshape(B, 1)
    bm = b_meta.reshape(1, E)
    bmv = b_mv.reshape(1, 1)
    bt = b_term.reshape(1, E)
    Wv_r = Wv[..., None]
    bv_r = bv.reshape(E, 1, 1)
    b1_r = b1.reshape(E, 1, HID)
    b2_r = b2.reshape(E, 1, ACT)

    grid = (E,)
    out = pl.pallas_call(
        _fused_body,
        grid=grid,
        in_specs=[
            pl.BlockSpec((B, OBS), lambda e: (0, 0)),
            pl.BlockSpec((B, 1), lambda e: (0, 0)),
            pl.BlockSpec((B, 1), lambda e: (0, 0)),
            pl.BlockSpec((OBS, E), lambda e: (0, 0)),
            pl.BlockSpec((1, E), lambda e: (0, 0)),
            pl.BlockSpec((OBS, 1), lambda e: (0, 0)),
            pl.BlockSpec((1, 1), lambda e: (0, 0)),
            pl.BlockSpec((OBS, E), lambda e: (0, 0)),
            pl.BlockSpec((1, E), lambda e: (0, 0)),
            pl.BlockSpec((1, OBS, HID), lambda e: (e, 0, 0)),
            pl.BlockSpec((1, 1, HID), lambda e: (e, 0, 0)),
            pl.BlockSpec((1, HID, ACT), lambda e: (e, 0, 0)),
            pl.BlockSpec((1, 1, ACT), lambda e: (e, 0, 0)),
            pl.BlockSpec((1, HID, 1), lambda e: (e, 0, 0)),
            pl.BlockSpec((1, 1, 1), lambda e: (e, 0, 0)),
        ],
        out_specs=[
            pl.BlockSpec((B, 1), lambda e: (0, 0)),
            pl.BlockSpec((B, 1), lambda e: (0, 0)),
            pl.BlockSpec((B, 1), lambda e: (0, 0)),
            pl.BlockSpec((B, 1), lambda e: (0, 0)),
            pl.BlockSpec((B, 1), lambda e: (0, 0)),
            pl.BlockSpec((B, 1), lambda e: (0, 0)),
            pl.BlockSpec((B, 1), lambda e: (0, 0)),
        ],
        out_shape=[
            jax.ShapeDtypeStruct((B, 1), jnp.int32),
            jax.ShapeDtypeStruct((B, 1), jnp.float32),
            jax.ShapeDtypeStruct((B, 1), jnp.float32),
            jax.ShapeDtypeStruct((B, 1), jnp.int32),
            jax.ShapeDtypeStruct((B, 1), jnp.float32),
            jax.ShapeDtypeStruct((B, 1), jnp.float32),
            jax.ShapeDtypeStruct((B, 1), jnp.float32),
        ],
        scratch_shapes=[pltpu.VMEM((B, 1), jnp.int32)],
    )(observation, dones_i, eo, W_meta, bm, W_mv, bmv, W_term, bt,
      W1, b1_r, W2, b2_r, Wv_r, bv_r)

    actions, values, log_probs, ma, mv, mlp, tp = out
    return (actions.reshape(B), values.reshape(B), log_probs.reshape(B),
            ma.reshape(B), mv.reshape(B), mlp.reshape(B), tp.reshape(B))
